# baseline (speedup 1.0000x reference)
import functools

import jax
import jax.numpy as jnp
from jax import lax
from jax.experimental import pallas as pl
from jax.experimental.pallas import tpu as pltpu
from jax.experimental.pallas import tpu_sc as plsc


def _build_type_table(table_p, b1, W2, b2, W3, b3, W4, b4, gamma, beta):
    R, half = table_p.shape
    d = W3.shape[0]

    def body(tab, b1r, W2r, b2r, W3r, b3r, W4r, b4r, gr, br, out):
        v = jnp.maximum(b1r[:], 0.0)
        v = jnp.dot(v, W2r[:], preferred_element_type=jnp.float32) + b2r[:]
        c = jnp.dot(v, W3r[half:, :], preferred_element_type=jnp.float32) + b3r[:]
        t = jnp.dot(tab[:], W3r[:half, :], preferred_element_type=jnp.float32) + c
        h2 = jnp.maximum(t, 0.0)
        o = jnp.dot(h2, W4r[:], preferred_element_type=jnp.float32) + b4r[:]
        mu = jnp.mean(o, axis=-1, keepdims=True)
        var = jnp.mean((o - mu) ** 2, axis=-1, keepdims=True)
        out[:] = (o - mu) / jnp.sqrt(var + 1e-5) * gr[:] + br[:]

    return pl.pallas_call(
        body,
        out_shape=jax.ShapeDtypeStruct((R, d), jnp.float32),
    )(
        table_p,
        b1.reshape(1, half),
        W2,
        b2.reshape(1, half),
        W3,
        b3.reshape(1, d),
        W4,
        b4.reshape(1, d),
        gamma.reshape(1, d),
        beta.reshape(1, d),
    )


def _sc_gather(ftab, idx):
    B = idx.shape[0]
    R, d = ftab.shape
    info = plsc.get_sparse_core_info()
    NC, NS = info.num_cores, info.num_subcores
    NW = NC * NS
    CH = 96

    assert B % 8 == 0
    g = B // 8
    big = -(-g // NW) * 8
    n_big = g % NW if g % NW else NW
    small = big - 8
    nfull = small // CH
    assert nfull == big // CH and nfull % 2 == 0 and nfull >= 4
    tail_big = big - nfull * CH
    tail_small = small - nfull * CH
    n_super = nfull // 2
    idx_pad = -(-big // 16) * 16

    mesh = plsc.VectorSubcoreMesh(core_axis_name="c", subcore_axis_name="s")

    @functools.partial(
        pl.kernel,
        mesh=mesh,
        out_type=jax.ShapeDtypeStruct((B, d), jnp.float32),
        scratch_types=[
            pltpu.VMEM((idx_pad,), jnp.int32),
            pltpu.VMEM((R * d,), jnp.float32),
            pltpu.VMEM((CH, d), jnp.float32),
            pltpu.VMEM((CH, d), jnp.float32),
            pltpu.VMEM((CH, d), jnp.float32),
            pltpu.VMEM((CH, d), jnp.float32),
            pltpu.SemaphoreType.DMA,
            pltpu.SemaphoreType.DMA,
            pltpu.SemaphoreType.DMA,
            pltpu.SemaphoreType.DMA,
            pltpu.SemaphoreType.DMA,
        ],
    )
    def gather_kernel(tab_hbm, tabf_hbm, idx_hbm, out_hbm, idx_v, tab_v,
                      sbuf0, sbuf1, cbuf0, cbuf1, gsem, ss0, ss1, cs0, cs1):
        wid = lax.axis_index("s") * NC + lax.axis_index("c")
        off = wid * big - jnp.maximum(wid - n_big, 0) * 8
        is_big = wid < n_big
        pltpu.sync_copy(tabf_hbm, tab_v)

        @pl.when(is_big)
        def _():
            pltpu.sync_copy(idx_hbm.at[pl.ds(off, big)],
                            idx_v.at[pl.ds(0, big)])

        @pl.when(jnp.logical_not(is_big))
        def _():
            pltpu.sync_copy(idx_hbm.at[pl.ds(off, small)],
                            idx_v.at[pl.ds(0, small)])

        def scat_start(c, buf, sem, rows=CH):
            pltpu.make_async_copy(
                buf.at[pl.ds(0, rows)],
                out_hbm.at[pl.ds(off + c * CH, rows)], sem).start()

        def scat_drain(buf, sem, rows=CH):
            pltpu.make_async_copy(
                buf.at[pl.ds(0, rows)],
                out_hbm.at[pl.ds(off, rows)], sem).wait()

        def cassemble(c, cbuf):
            def group16(gi):
                iv = idx_v[pl.ds(c * CH + gi * 16, 16)]
                for r in range(16):
                    src = iv[r] * d
                    vals = [tab_v[pl.ds(src + k * 16, 16)]
                            for k in range(d // 16)]
                    for k in range(d // 16):
                        cbuf[gi * 16 + r, pl.ds(k * 16, 16)] = vals[k]

            def group_body(gi, carry):
                group16(gi)
                return carry

            lax.fori_loop(0, CH // 16, group_body, 0)

        def super_iter(si, sbuf, ssem, cbuf, csem, with_drain):
            s_c, c_c = 2 * si, 2 * si + 1
            if with_drain:
                scat_drain(sbuf, ssem)
                scat_drain(cbuf, csem)
            h = pltpu.async_copy(
                tab_hbm.at[idx_v.at[pl.ds(s_c * CH, CH)]], sbuf, gsem)
            cassemble(c_c, cbuf)
            scat_start(c_c, cbuf, csem)
            h.wait()
            scat_start(s_c, sbuf, ssem)

        super_iter(0, sbuf0, ss0, cbuf0, cs0, False)
        super_iter(1, sbuf1, ss1, cbuf1, cs1, False)

        def pair_body(j, carry):
            super_iter(2 * j, sbuf0, ss0, cbuf0, cs0, True)
            super_iter(2 * j + 1, sbuf1, ss1, cbuf1, cs1, True)
            return carry

        lax.fori_loop(1, n_super // 2, pair_body, 0)

        def do_tail(tail):
            scat_drain(sbuf0, ss0)
            if tail:
                pltpu.async_copy(
                    tab_hbm.at[idx_v.at[pl.ds(nfull * CH, tail)]],
                    sbuf0.at[pl.ds(0, tail)], gsem).wait()
                scat_start(nfull, sbuf0, ss0, tail)
                scat_drain(sbuf0, ss0, tail)
            scat_drain(cbuf0, cs0)
            scat_drain(sbuf1, ss1)
            scat_drain(cbuf1, cs1)

        @pl.when(is_big)
        def _():
            do_tail(tail_big)

        @pl.when(jnp.logical_not(is_big))
        def _():
            do_tail(tail_small)

    return gather_kernel(ftab, ftab.reshape(-1), idx)


def kernel(atom_types, n_atoms, table, W1, b1, W2, b2, W3, b3, W4, b4, gamma, beta):
    n_types, half = table.shape
    B = atom_types.shape[0]
    R = -(-n_types // 8) * 8
    table_p = jnp.pad(table, ((0, R - n_types), (0, 0)))
    ftab = _build_type_table(table_p, b1, W2, b2, W3, b3, W4, b4, gamma, beta)
    idx = atom_types.astype(jnp.int32)
    return _sc_gather(ftab, idx)


# speedup vs baseline: 1.0814x; 1.0814x over previous
import functools

import jax
import jax.numpy as jnp
from jax import lax
from jax.experimental import pallas as pl
from jax.experimental.pallas import tpu as pltpu
from jax.experimental.pallas import tpu_sc as plsc


def _build_type_table(table_p, b1, W2, b2, W3, b3, W4, b4, gamma, beta):
    R, half = table_p.shape
    d = W3.shape[0]

    def body(tab, b1r, W2r, b2r, W3r, b3r, W4r, b4r, gr, br, out):
        v = jnp.maximum(b1r[:], 0.0)
        v = jnp.dot(v, W2r[:], preferred_element_type=jnp.float32) + b2r[:]
        c = jnp.dot(v, W3r[half:, :], preferred_element_type=jnp.float32) + b3r[:]
        t = jnp.dot(tab[:], W3r[:half, :], preferred_element_type=jnp.float32) + c
        h2 = jnp.maximum(t, 0.0)
        o = jnp.dot(h2, W4r[:], preferred_element_type=jnp.float32) + b4r[:]
        mu = jnp.mean(o, axis=-1, keepdims=True)
        var = jnp.mean((o - mu) ** 2, axis=-1, keepdims=True)
        out[:] = (o - mu) / jnp.sqrt(var + 1e-5) * gr[:] + br[:]

    return pl.pallas_call(
        body,
        out_shape=jax.ShapeDtypeStruct((R, d), jnp.float32),
    )(
        table_p,
        b1.reshape(1, half),
        W2,
        b2.reshape(1, half),
        W3,
        b3.reshape(1, d),
        W4,
        b4.reshape(1, d),
        gamma.reshape(1, d),
        beta.reshape(1, d),
    )


def _tc_fill(o1, ftab, idx_tail, first_block, blk):
    B, d = o1.shape
    R = ftab.shape[0]
    nb = idx_tail.shape[0]

    def body(o1_any, ftab_ref, idx_ref, out_ref):
        del o1_any
        idxb = idx_ref[0, 0, :]
        oh = (idxb[:, None] ==
              lax.broadcasted_iota(jnp.int32, (blk, R), 1)).astype(jnp.float32)
        out_ref[:] = jnp.dot(oh, ftab_ref[:],
                             preferred_element_type=jnp.float32)

    return pl.pallas_call(
        body,
        grid=(nb,),
        in_specs=[
            pl.BlockSpec(memory_space=pl.ANY),
            pl.BlockSpec((R, d), lambda i: (0, 0)),
            pl.BlockSpec((1, 1, blk), lambda i: (i, 0, 0)),
        ],
        out_specs=pl.BlockSpec((blk, d), lambda i: (first_block + i, 0)),
        out_shape=jax.ShapeDtypeStruct((B, d), jnp.float32),
        input_output_aliases={0: 0},
    )(o1, ftab, idx_tail)


def _sc_gather(ftab, idx, S):
    B = idx.shape[0]
    R, d = ftab.shape
    info = plsc.get_sparse_core_info()
    NC, NS = info.num_cores, info.num_subcores
    NW = NC * NS
    CH = 96

    assert S % 8 == 0
    g = S // 8
    big = -(-g // NW) * 8
    n_big = g % NW if g % NW else NW
    small = big - 8
    nfull = small // CH
    assert nfull == big // CH and nfull % 2 == 0 and nfull >= 4
    tail_big = big - nfull * CH
    tail_small = small - nfull * CH
    n_super = nfull // 2
    idx_pad = -(-big // 16) * 16

    mesh = plsc.VectorSubcoreMesh(core_axis_name="c", subcore_axis_name="s")

    @functools.partial(
        pl.kernel,
        mesh=mesh,
        out_type=jax.ShapeDtypeStruct((B, d), jnp.float32),
        scratch_types=[
            pltpu.VMEM((idx_pad,), jnp.int32),
            pltpu.VMEM((R * d,), jnp.float32),
            pltpu.VMEM((CH, d), jnp.float32),
            pltpu.VMEM((CH, d), jnp.float32),
            pltpu.VMEM((CH, d), jnp.float32),
            pltpu.VMEM((CH, d), jnp.float32),
            pltpu.SemaphoreType.DMA,
            pltpu.SemaphoreType.DMA,
            pltpu.SemaphoreType.DMA,
            pltpu.SemaphoreType.DMA,
            pltpu.SemaphoreType.DMA,
        ],
    )
    def gather_kernel(tab_hbm, tabf_hbm, idx_hbm, out_hbm, idx_v, tab_v,
                      sbuf0, sbuf1, cbuf0, cbuf1, gsem, ss0, ss1, cs0, cs1):
        wid = lax.axis_index("s") * NC + lax.axis_index("c")
        off = wid * big - jnp.maximum(wid - n_big, 0) * 8
        is_big = wid < n_big
        pltpu.sync_copy(tabf_hbm, tab_v)

        @pl.when(is_big)
        def _():
            pltpu.sync_copy(idx_hbm.at[pl.ds(off, big)],
                            idx_v.at[pl.ds(0, big)])

        @pl.when(jnp.logical_not(is_big))
        def _():
            pltpu.sync_copy(idx_hbm.at[pl.ds(off, small)],
                            idx_v.at[pl.ds(0, small)])

        def scat_start(c, buf, sem, rows=CH):
            pltpu.make_async_copy(
                buf.at[pl.ds(0, rows)],
                out_hbm.at[pl.ds(off + c * CH, rows)], sem).start()

        def scat_drain(buf, sem, rows=CH):
            pltpu.make_async_copy(
                buf.at[pl.ds(0, rows)],
                out_hbm.at[pl.ds(off, rows)], sem).wait()

        def cassemble(c, cbuf):
            def group16(gi):
                iv = idx_v[pl.ds(c * CH + gi * 16, 16)]
                for r in range(16):
                    src = iv[r] * d
                    vals = [tab_v[pl.ds(src + k * 16, 16)]
                            for k in range(d // 16)]
                    for k in range(d // 16):
                        cbuf[gi * 16 + r, pl.ds(k * 16, 16)] = vals[k]

            def group_body(gi, carry):
                group16(gi)
                return carry

            lax.fori_loop(0, CH // 16, group_body, 0)

        def super_iter(si, sbuf, ssem, cbuf, csem, with_drain):
            s_c, c_c = 2 * si, 2 * si + 1
            if with_drain:
                scat_drain(sbuf, ssem)
                scat_drain(cbuf, csem)
            h = pltpu.async_copy(
                tab_hbm.at[idx_v.at[pl.ds(s_c * CH, CH)]], sbuf, gsem)
            cassemble(c_c, cbuf)
            scat_start(c_c, cbuf, csem)
            h.wait()
            scat_start(s_c, sbuf, ssem)

        super_iter(0, sbuf0, ss0, cbuf0, cs0, False)
        super_iter(1, sbuf1, ss1, cbuf1, cs1, False)

        def pair_body(j, carry):
            super_iter(2 * j, sbuf0, ss0, cbuf0, cs0, True)
            super_iter(2 * j + 1, sbuf1, ss1, cbuf1, cs1, True)
            return carry

        lax.fori_loop(1, n_super // 2, pair_body, 0)

        def do_tail(tail):
            scat_drain(sbuf0, ss0)
            if tail:
                pltpu.async_copy(
                    tab_hbm.at[idx_v.at[pl.ds(nfull * CH, tail)]],
                    sbuf0.at[pl.ds(0, tail)], gsem).wait()
                scat_start(nfull, sbuf0, ss0, tail)
                scat_drain(sbuf0, ss0, tail)
            scat_drain(cbuf0, cs0)
            scat_drain(sbuf1, ss1)
            scat_drain(cbuf1, cs1)

        @pl.when(is_big)
        def _():
            do_tail(tail_big)

        @pl.when(jnp.logical_not(is_big))
        def _():
            do_tail(tail_small)

    return gather_kernel(ftab, ftab.reshape(-1), idx)


def kernel(atom_types, n_atoms, table, W1, b1, W2, b2, W3, b3, W4, b4, gamma, beta):
    n_types, half = table.shape
    B = atom_types.shape[0]
    R = -(-n_types // 8) * 8
    table_p = jnp.pad(table, ((0, R - n_types), (0, 0)))
    ftab = _build_type_table(table_p, b1, W2, b2, W3, b3, W4, b4, gamma, beta)
    idx = atom_types.astype(jnp.int32)
    S, blk = 62400, 800
    nb = (B - S) // blk
    assert nb * blk == B - S and S % blk == 0
    o1 = _sc_gather(ftab, idx, S)
    idx_tail = idx[S:].reshape(nb, 1, blk)
    return _tc_fill(o1, ftab, idx_tail, S // blk, blk)


# speedup vs baseline: 1.1118x; 1.0281x over previous
import functools

import jax
import jax.numpy as jnp
from jax import lax
from jax.experimental import pallas as pl
from jax.experimental.pallas import tpu as pltpu
from jax.experimental.pallas import tpu_sc as plsc


def _build_type_table(table_p, b1, W2, b2, W3, b3, W4, b4, gamma, beta):
    R, half = table_p.shape
    d = W3.shape[0]

    def body(tab, b1r, W2r, b2r, W3r, b3r, W4r, b4r, gr, br, out):
        v = jnp.maximum(b1r[:], 0.0)
        v = jnp.dot(v, W2r[:], preferred_element_type=jnp.float32) + b2r[:]
        c = jnp.dot(v, W3r[half:, :], preferred_element_type=jnp.float32) + b3r[:]
        t = jnp.dot(tab[:], W3r[:half, :], preferred_element_type=jnp.float32) + c
        h2 = jnp.maximum(t, 0.0)
        o = jnp.dot(h2, W4r[:], preferred_element_type=jnp.float32) + b4r[:]
        mu = jnp.mean(o, axis=-1, keepdims=True)
        var = jnp.mean((o - mu) ** 2, axis=-1, keepdims=True)
        out[:] = (o - mu) / jnp.sqrt(var + 1e-5) * gr[:] + br[:]

    return pl.pallas_call(
        body,
        out_shape=jax.ShapeDtypeStruct((R, d), jnp.float32),
    )(
        table_p,
        b1.reshape(1, half),
        W2,
        b2.reshape(1, half),
        W3,
        b3.reshape(1, d),
        W4,
        b4.reshape(1, d),
        gamma.reshape(1, d),
        beta.reshape(1, d),
    )


def _tc_fill(o1, ftab, idx_tail, first_block, blk):
    B, d = o1.shape
    R = ftab.shape[0]
    nb = idx_tail.shape[0]

    def body(o1_any, ftab_ref, idx_ref, out_ref):
        del o1_any
        idxb = idx_ref[0, 0, :]
        oh = (idxb[:, None] ==
              lax.broadcasted_iota(jnp.int32, (blk, R), 1)).astype(jnp.float32)
        out_ref[:] = jnp.dot(oh, ftab_ref[:],
                             precision=lax.Precision.HIGHEST,
                             preferred_element_type=jnp.float32)

    return pl.pallas_call(
        body,
        grid=(nb,),
        in_specs=[
            pl.BlockSpec(memory_space=pl.ANY),
            pl.BlockSpec((R, d), lambda i: (0, 0)),
            pl.BlockSpec((1, 1, blk), lambda i: (i, 0, 0)),
        ],
        out_specs=pl.BlockSpec((blk, d), lambda i: (first_block + i, 0)),
        out_shape=jax.ShapeDtypeStruct((B, d), jnp.float32),
        input_output_aliases={0: 0},
    )(o1, ftab, idx_tail)


def _sc_gather(ftab, idx, S):
    B = idx.shape[0]
    R, d = ftab.shape
    info = plsc.get_sparse_core_info()
    NC, NS = info.num_cores, info.num_subcores
    NW = NC * NS
    CH = 96

    assert S % 8 == 0
    g = S // 8
    big = -(-g // NW) * 8
    n_big = g % NW if g % NW else NW
    small = big - 8
    nfull = small // CH
    assert nfull == big // CH and nfull % 2 == 0 and nfull >= 4
    tail_big = big - nfull * CH
    tail_small = small - nfull * CH
    n_super = nfull // 2
    idx_pad = -(-big // 16) * 16

    mesh = plsc.VectorSubcoreMesh(core_axis_name="c", subcore_axis_name="s")

    @functools.partial(
        pl.kernel,
        mesh=mesh,
        out_type=jax.ShapeDtypeStruct((B, d), jnp.float32),
        scratch_types=[
            pltpu.VMEM((idx_pad,), jnp.int32),
            pltpu.VMEM((R * d,), jnp.float32),
            pltpu.VMEM((CH, d), jnp.float32),
            pltpu.VMEM((CH, d), jnp.float32),
            pltpu.VMEM((CH, d), jnp.float32),
            pltpu.VMEM((CH, d), jnp.float32),
            pltpu.SemaphoreType.DMA,
            pltpu.SemaphoreType.DMA,
            pltpu.SemaphoreType.DMA,
            pltpu.SemaphoreType.DMA,
            pltpu.SemaphoreType.DMA,
        ],
    )
    def gather_kernel(tab_hbm, tabf_hbm, idx_hbm, out_hbm, idx_v, tab_v,
                      sbuf0, sbuf1, cbuf0, cbuf1, gsem, ss0, ss1, cs0, cs1):
        wid = lax.axis_index("s") * NC + lax.axis_index("c")
        off = wid * big - jnp.maximum(wid - n_big, 0) * 8
        is_big = wid < n_big
        pltpu.sync_copy(tabf_hbm, tab_v)

        @pl.when(is_big)
        def _():
            pltpu.sync_copy(idx_hbm.at[pl.ds(off, big)],
                            idx_v.at[pl.ds(0, big)])

        @pl.when(jnp.logical_not(is_big))
        def _():
            pltpu.sync_copy(idx_hbm.at[pl.ds(off, small)],
                            idx_v.at[pl.ds(0, small)])

        def scat_start(c, buf, sem, rows=CH):
            pltpu.make_async_copy(
                buf.at[pl.ds(0, rows)],
                out_hbm.at[pl.ds(off + c * CH, rows)], sem).start()

        def scat_drain(buf, sem, rows=CH):
            pltpu.make_async_copy(
                buf.at[pl.ds(0, rows)],
                out_hbm.at[pl.ds(off, rows)], sem).wait()

        def cassemble(c, cbuf):
            def group16(gi):
                iv = idx_v[pl.ds(c * CH + gi * 16, 16)]
                for r in range(16):
                    src = iv[r] * d
                    vals = [tab_v[pl.ds(src + k * 16, 16)]
                            for k in range(d // 16)]
                    for k in range(d // 16):
                        cbuf[gi * 16 + r, pl.ds(k * 16, 16)] = vals[k]

            def group_body(gi, carry):
                group16(gi)
                return carry

            lax.fori_loop(0, CH // 16, group_body, 0)

        def super_iter(si, sbuf, ssem, cbuf, csem, with_drain):
            s_c, c_c = 2 * si, 2 * si + 1
            if with_drain:
                scat_drain(sbuf, ssem)
                scat_drain(cbuf, csem)
            h = pltpu.async_copy(
                tab_hbm.at[idx_v.at[pl.ds(s_c * CH, CH)]], sbuf, gsem)
            cassemble(c_c, cbuf)
            scat_start(c_c, cbuf, csem)
            h.wait()
            scat_start(s_c, sbuf, ssem)

        super_iter(0, sbuf0, ss0, cbuf0, cs0, False)
        super_iter(1, sbuf1, ss1, cbuf1, cs1, False)

        def pair_body(j, carry):
            super_iter(2 * j, sbuf0, ss0, cbuf0, cs0, True)
            super_iter(2 * j + 1, sbuf1, ss1, cbuf1, cs1, True)
            return carry

        lax.fori_loop(1, n_super // 2, pair_body, 0)

        def do_tail(tail):
            scat_drain(sbuf0, ss0)
            if tail:
                pltpu.async_copy(
                    tab_hbm.at[idx_v.at[pl.ds(nfull * CH, tail)]],
                    sbuf0.at[pl.ds(0, tail)], gsem).wait()
                scat_start(nfull, sbuf0, ss0, tail)
                scat_drain(sbuf0, ss0, tail)
            scat_drain(cbuf0, cs0)
            scat_drain(sbuf1, ss1)
            scat_drain(cbuf1, cs1)

        @pl.when(is_big)
        def _():
            do_tail(tail_big)

        @pl.when(jnp.logical_not(is_big))
        def _():
            do_tail(tail_small)

    return gather_kernel(ftab, ftab.reshape(-1), idx)


def kernel(atom_types, n_atoms, table, W1, b1, W2, b2, W3, b3, W4, b4, gamma, beta):
    n_types, half = table.shape
    B = atom_types.shape[0]
    R = -(-n_types // 8) * 8
    table_p = jnp.pad(table, ((0, R - n_types), (0, 0)))
    ftab = _build_type_table(table_p, b1, W2, b2, W3, b3, W4, b4, gamma, beta)
    idx = atom_types.astype(jnp.int32)
    S, blk = 56000, 800
    nb = (B - S) // blk
    assert nb * blk == B - S and S % blk == 0
    o1 = _sc_gather(ftab, idx, S)
    idx_tail = idx[S:].reshape(nb, 1, blk)
    return _tc_fill(o1, ftab, idx_tail, S // blk, blk)
